# baseline (speedup 1.0000x reference)
import functools

import jax
import jax.numpy as jnp
from jax import lax
from jax.experimental import pallas as pl
from jax.experimental.pallas import tpu as pltpu
from jax.experimental.pallas import tpu_sc as plsc

B, T, ENC, HID = 16, 64, 1024, 1024
L = 16
NG = T // L
NCH = ENC // L

_LN2 = 0.6931471805599453
_SQRT2 = 1.4142135623730951


def _log_f32(x):
    bits = plsc.bitcast(x, jnp.int32)
    e_raw = (bits >> 23) - 127
    man = plsc.bitcast((bits & 0x007FFFFF) | 0x3F800000, jnp.float32)
    adj = man > _SQRT2
    man = jnp.where(adj, man * 0.5, man)
    e_f = (e_raw + jnp.where(adj, 1, 0)).astype(jnp.float32)
    t = (man - 1.0) / (man + 1.0)
    t2 = t * t
    ln_man = t * (2.0 + t2 * (0.66666667 + t2 * (0.4 + t2 * 0.28571429)))
    return e_f * _LN2 + ln_man


NBLK = 2
BLKC = ENC // NBLK


def _sc_body(x_hbm, mask_hbm, wx_hbm, out_hbm,
             xb, wv, mv, stage, out_v, redmat, shared,
             sem0, sem1, sem2, sem3):
    cid = lax.axis_index("c")
    sid = lax.axis_index("s")
    sems = [sem0, sem1, sem2, sem3][:NBLK]

    @pl.when(cid == 0)
    def _per_batch():
        b = sid
        copies = [
            pltpu.make_async_copy(
                x_hbm.at[b, :, pl.ds(j * BLKC, BLKC)],
                xb.at[:, pl.ds(j * BLKC, BLKC)],
                sems[j])
            for j in range(NBLK)
        ]
        for c in copies:
            c.start()
        pltpu.sync_copy(wx_hbm, wv)
        pltpu.sync_copy(mask_hbm.at[b], mv)

        lane = lax.iota(jnp.int32, L)
        row_idx = [lane + L * g for g in range(NG)]

        def dot_step(cc, accs):
            wc = wv[pl.ds(cc * L, L)]
            base = jnp.broadcast_to(cc * L, (L,)).astype(jnp.int32)
            accs = [list(a) for a in accs]
            for d in range(L):
                perm = (lane + d) & (L - 1)
                col = base + perm
                w = wc.at[perm].get(mode="promise_in_bounds",
                                    unique_indices=True)
                for g in range(NG):
                    accs[g][d & 1] = accs[g][d & 1] + w * plsc.load_gather(
                        xb, [row_idx[g], col])
            return tuple(tuple(a) for a in accs)

        zeros = jnp.zeros((L,), jnp.float32)
        accs = tuple((zeros, zeros) for _ in range(NG))
        for j in range(NBLK):
            copies[j].wait()
            accs = lax.fori_loop(j * (BLKC // L), (j + 1) * (BLKC // L),
                                 dot_step, accs, unroll=8)
        bk = [accs[g][0] + accs[g][1] for g in range(NG)]

        length = jnp.zeros((), jnp.int32)
        for g in range(NG):
            length = length + jnp.sum(mv[pl.ds(L * g, L)])

        m = jnp.float32(-jnp.inf)
        for g in range(NG):
            m = jnp.maximum(m, jnp.max(bk[g]))

        e = [jnp.where(row_idx[g] < length, jnp.exp(bk[g] - m), 0.0)
             for g in range(NG)]

        suf = [None] * NG
        carry = jnp.float32(0.0)
        for g in range(NG - 1, -1, -1):
            rc = lax.rev(plsc.cumsum(lax.rev(e[g], (0,))), (0,))
            suf[g] = rc + carry
            carry = carry + jnp.sum(e[g])

        loss_vec = jnp.zeros((L,), jnp.float32)
        for g in range(NG):
            s_ok = (row_idx[g] >= 3) & (row_idx[g] < length)
            term = m + _log_f32(suf[g]) - bk[g]
            loss_vec = loss_vec + jnp.where(s_ok, term, 0.0)

        count = jnp.maximum(length - 3, 0).astype(jnp.float32)

        stage[pl.ds(0, L)] = loss_vec
        stage[pl.ds(L, L)] = jnp.broadcast_to(count * (1.0 / L), (L,))
        pltpu.sync_copy(stage, shared.at[b])

    plsc.subcore_barrier()

    @pl.when((cid == 0) & (sid == 0))
    def _reduce():
        pltpu.sync_copy(shared, redmat)
        acc_l = jnp.zeros((L,), jnp.float32)
        acc_c = jnp.zeros((L,), jnp.float32)
        for b in range(B):
            acc_l = acc_l + redmat[b, pl.ds(0, L)]
            acc_c = acc_c + redmat[b, pl.ds(L, L)]
        num = jnp.broadcast_to(jnp.sum(acc_l), (L,))
        den = jnp.broadcast_to(jnp.sum(acc_c), (L,))
        out_v[...] = num / den
        pltpu.sync_copy(out_v, out_hbm)


@functools.partial(jax.jit, static_argnums=())
def _sc_loss(x, mask, wx):
    mesh = plsc.VectorSubcoreMesh(core_axis_name="c", subcore_axis_name="s",
                                  num_cores=2, num_subcores=16)
    run = pl.kernel(
        _sc_body,
        out_type=jax.ShapeDtypeStruct((L,), jnp.float32),
        mesh=mesh,
        compiler_params=pltpu.CompilerParams(use_tc_tiling_on_sc=False,
                                             needs_layout_passes=False),
        scratch_types=[
            pltpu.VMEM((T, ENC), jnp.float32),
            pltpu.VMEM((ENC,), jnp.float32),
            pltpu.VMEM((T,), jnp.int32),
            pltpu.VMEM((2 * L,), jnp.float32),
            pltpu.VMEM((L,), jnp.float32),
            pltpu.VMEM((B, 2 * L), jnp.float32),
            pltpu.VMEM_SHARED((B, 2 * L), jnp.float32),
            pltpu.SemaphoreType.DMA,
            pltpu.SemaphoreType.DMA,
            pltpu.SemaphoreType.DMA,
            pltpu.SemaphoreType.DMA,
        ],
    )
    return run(x, mask, wx)


def kernel(encoder_output, mask, W_ih, W_hh, b_ih, b_hh, W_fc, b_fc):
    del W_ih, W_hh, b_ih, b_hh, b_fc
    wx = W_fc[0, HID:]
    out = _sc_loss(encoder_output, mask, wx)
    return out[0]


# speedup vs baseline: 1.5102x; 1.5102x over previous
import jax
import jax.numpy as jnp
from jax import lax
from jax.experimental import pallas as pl
from jax.experimental.pallas import tpu as pltpu
from jax.experimental.pallas import tpu_sc as plsc

B, T, ENC, HID = 16, 64, 1024, 1024
L = 16
NG = T // L

_LN2 = 0.6931471805599453
_SQRT2 = 1.4142135623730951


def _bk_kernel(x_ref, wx_ref, out_ref):
    x = x_ref[...]
    wx = wx_ref[...]
    out_ref[...] = jnp.sum(x * wx[None, :, :], axis=-1)


def _log_f32(x):
    bits = plsc.bitcast(x, jnp.int32)
    e_raw = (bits >> 23) - 127
    man = plsc.bitcast((bits & 0x007FFFFF) | 0x3F800000, jnp.float32)
    adj = man > _SQRT2
    man = jnp.where(adj, man * 0.5, man)
    e_f = (e_raw + jnp.where(adj, 1, 0)).astype(jnp.float32)
    t = (man - 1.0) / (man + 1.0)
    t2 = t * t
    ln_man = t * (2.0 + t2 * (0.66666667 + t2 * (0.4 + t2 * 0.28571429)))
    return e_f * _LN2 + ln_man


def _sc_body(bk_hbm, mask_hbm, out_hbm,
             bv, mv, stage, out_v, redmat, shared):
    cid = lax.axis_index("c")
    sid = lax.axis_index("s")

    @pl.when(cid == 0)
    def _per_batch():
        b = sid
        pltpu.sync_copy(bk_hbm.at[b], bv)
        pltpu.sync_copy(mask_hbm.at[b], mv)

        lane = lax.iota(jnp.int32, L)
        row_idx = [lane + L * g for g in range(NG)]
        bk = [bv[pl.ds(L * g, L)] for g in range(NG)]

        length = jnp.zeros((), jnp.int32)
        for g in range(NG):
            length = length + jnp.sum(mv[pl.ds(L * g, L)])

        m = jnp.float32(-jnp.inf)
        for g in range(NG):
            m = jnp.maximum(m, jnp.max(bk[g]))

        e = [jnp.where(row_idx[g] < length, jnp.exp(bk[g] - m), 0.0)
             for g in range(NG)]

        suf = [None] * NG
        carry = jnp.float32(0.0)
        for g in range(NG - 1, -1, -1):
            rc = lax.rev(plsc.cumsum(lax.rev(e[g], (0,))), (0,))
            suf[g] = rc + carry
            carry = carry + jnp.sum(e[g])

        loss_vec = jnp.zeros((L,), jnp.float32)
        for g in range(NG):
            s_ok = (row_idx[g] >= 3) & (row_idx[g] < length)
            term = m + _log_f32(suf[g]) - bk[g]
            loss_vec = loss_vec + jnp.where(s_ok, term, 0.0)

        count = jnp.maximum(length - 3, 0).astype(jnp.float32)

        stage[pl.ds(0, L)] = loss_vec
        stage[pl.ds(L, L)] = jnp.broadcast_to(count * (1.0 / L), (L,))
        pltpu.sync_copy(stage, shared.at[b])

    plsc.subcore_barrier()

    @pl.when((cid == 0) & (sid == 0))
    def _reduce():
        pltpu.sync_copy(shared, redmat)
        acc_l = jnp.zeros((L,), jnp.float32)
        acc_c = jnp.zeros((L,), jnp.float32)
        for b in range(B):
            acc_l = acc_l + redmat[b, pl.ds(0, L)]
            acc_c = acc_c + redmat[b, pl.ds(L, L)]
        num = jnp.broadcast_to(jnp.sum(acc_l), (L,))
        den = jnp.broadcast_to(jnp.sum(acc_c), (L,))
        out_v[...] = num / den
        pltpu.sync_copy(out_v, out_hbm)


@jax.jit
def _hybrid_loss(x, mask, wx2d):
    bk = pl.pallas_call(
        _bk_kernel,
        out_shape=jax.ShapeDtypeStruct((B, T), jnp.float32),
    )(x, wx2d)
    mesh = plsc.VectorSubcoreMesh(core_axis_name="c", subcore_axis_name="s",
                                  num_cores=2, num_subcores=16)
    run = pl.kernel(
        _sc_body,
        out_type=jax.ShapeDtypeStruct((L,), jnp.float32),
        mesh=mesh,
        compiler_params=pltpu.CompilerParams(use_tc_tiling_on_sc=False,
                                             needs_layout_passes=False),
        scratch_types=[
            pltpu.VMEM((T,), jnp.float32),
            pltpu.VMEM((T,), jnp.int32),
            pltpu.VMEM((2 * L,), jnp.float32),
            pltpu.VMEM((L,), jnp.float32),
            pltpu.VMEM((B, 2 * L), jnp.float32),
            pltpu.VMEM_SHARED((B, 2 * L), jnp.float32),
        ],
    )
    return run(bk, mask)


def kernel(encoder_output, mask, W_ih, W_hh, b_ih, b_hh, W_fc, b_fc):
    del W_ih, W_hh, b_ih, b_hh, b_fc
    wx2d = W_fc[:, HID:]
    out = _hybrid_loss(encoder_output, mask, wx2d)
    return out[0]
